# baseline (device time: 23691 ns/iter reference)
import functools

import jax
import jax.numpy as jnp
from jax import lax
from jax.experimental import pallas as pl
from jax.experimental.pallas import tpu as pltpu

V_PER_SHARD = 4096
NCHUNK = 32


def kernel(ids, E):
    T = ids.shape[0]
    D = E.shape[1]

    my_x = lax.axis_index("x")
    local = ids - my_x * V_PER_SHARD
    owned = (local >= 0) & (local < V_PER_SHARD)
    safe = jnp.where(owned, local, 0)
    part = jnp.take(E, safe, axis=0)
    mask = owned.astype(jnp.int32)[:, None]

    R = T // NCHUNK
    NSEM = NCHUNK + 1

    def body(mask_ref, part_ref, out_ref, sbuf_ref, rbufa_ref, rbufb_ref,
             csum_ref, creca_ref, crecb_ref,
             send_a, recv_a, send_b, recv_b):
        x = lax.axis_index("x")
        y = lax.axis_index("y")
        z = lax.axis_index("z")
        nbr = (1 - x, y, z)

        sbuf_ref[:, :] = part_ref[:, :].astype(jnp.bfloat16)
        ci = lax.broadcasted_iota(jnp.int32, (NCHUNK, T), 0)
        ti = lax.broadcasted_iota(jnp.int32, (NCHUNK, T), 1)
        G = (ti // R == ci).astype(jnp.bfloat16)
        gt_t = lax.broadcasted_iota(jnp.int32, (T, NCHUNK), 0)
        gt_c = lax.broadcasted_iota(jnp.int32, (T, NCHUNK), 1)
        GT = (gt_t // R == gt_c).astype(jnp.float32)
        dims = (((1,), (0,)), ((), ()))

        def chunk_sums(buf_bf16):
            return lax.dot_general(
                G, buf_bf16, dims, preferred_element_type=jnp.float32
            )

        csum_ref[:, :] = chunk_sums(sbuf_ref[:, :])

        barrier_sem = pltpu.get_barrier_semaphore()
        pl.semaphore_signal(
            barrier_sem, inc=1, device_id=nbr,
            device_id_type=pl.DeviceIdType.MESH,
        )
        pl.semaphore_wait(barrier_sem, 1)

        def round_rdmas(rbuf_ref, crec_ref, send_sems, recv_sems):
            rs = [
                pltpu.make_async_remote_copy(
                    src_ref=sbuf_ref.at[pl.ds(c * R, R)],
                    dst_ref=rbuf_ref.at[pl.ds(c * R, R)],
                    send_sem=send_sems.at[c],
                    recv_sem=recv_sems.at[c],
                    device_id=nbr,
                    device_id_type=pl.DeviceIdType.MESH,
                )
                for c in range(NCHUNK)
            ]
            rs.append(pltpu.make_async_remote_copy(
                src_ref=csum_ref,
                dst_ref=crec_ref,
                send_sem=send_sems.at[NCHUNK],
                recv_sem=recv_sems.at[NCHUNK],
                device_id=nbr,
                device_id_type=pl.DeviceIdType.MESH,
            ))
            return rs

        rdmas = round_rdmas(rbufa_ref, creca_ref, send_a, recv_a)
        rdmas += round_rdmas(rbufb_ref, crecb_ref, send_b, recv_b)
        for r in rdmas:
            r.start()
        for r in rdmas:
            r.wait()

        def bad_rows(recomputed, truth):
            neq = (recomputed != truth).astype(jnp.float32)
            bad_chunk = jnp.sum(neq, axis=1, keepdims=True)
            return lax.dot_general(
                GT, bad_chunk, dims, preferred_element_type=jnp.float32
            ) > 0

        bad_a = bad_rows(chunk_sums(rbufa_ref[:, :]), creca_ref[:, :])
        remote = jnp.where(
            bad_a, rbufb_ref[:, :], rbufa_ref[:, :]
        ).astype(jnp.float32)

        bad_p = bad_rows(
            chunk_sums(part_ref[:, :].astype(jnp.bfloat16)), csum_ref[:, :]
        )
        own = jnp.where(
            bad_p, sbuf_ref[:, :].astype(jnp.float32), part_ref[:, :]
        )

        m = mask_ref[:, :] > 0
        out_ref[:, :] = jnp.where(m, own, remote)

        @functools.partial(
            pl.run_scoped, second_barrier=pltpu.SemaphoreType.REGULAR
        )
        def _(second_barrier):
            pl.semaphore_signal(
                second_barrier, inc=1, device_id=nbr,
                device_id_type=pl.DeviceIdType.MESH,
            )
            pl.semaphore_wait(second_barrier, 1)

    return pl.pallas_call(
        body,
        out_shape=jax.ShapeDtypeStruct((T, D), jnp.float32),
        in_specs=[
            pl.BlockSpec(memory_space=pltpu.VMEM),
            pl.BlockSpec(memory_space=pltpu.VMEM),
        ],
        out_specs=pl.BlockSpec(memory_space=pltpu.VMEM),
        scratch_shapes=[
            pltpu.VMEM((T, D), jnp.bfloat16),
            pltpu.VMEM((T, D), jnp.bfloat16),
            pltpu.VMEM((T, D), jnp.bfloat16),
            pltpu.VMEM((NCHUNK, D), jnp.float32),
            pltpu.VMEM((NCHUNK, D), jnp.float32),
            pltpu.VMEM((NCHUNK, D), jnp.float32),
            pltpu.SemaphoreType.DMA((NSEM,)),
            pltpu.SemaphoreType.DMA((NSEM,)),
            pltpu.SemaphoreType.DMA((NSEM,)),
            pltpu.SemaphoreType.DMA((NSEM,)),
        ],
        compiler_params=pltpu.CompilerParams(collective_id=0),
    )(mask, part)


# device time: 22826 ns/iter; 1.0379x vs baseline; 1.0379x over previous
import functools

import jax
import jax.numpy as jnp
from jax import lax
from jax.experimental import pallas as pl
from jax.experimental.pallas import tpu as pltpu

V_PER_SHARD = 4096
NCHUNK = 32


def kernel(ids, E):
    T = ids.shape[0]
    D = E.shape[1]

    my_x = lax.axis_index("x")
    local = ids - my_x * V_PER_SHARD
    owned = (local >= 0) & (local < V_PER_SHARD)
    safe = jnp.where(owned, local, 0)
    part = jnp.take(E, safe, axis=0)
    mask = owned.astype(jnp.int32)[:, None]

    R = T // NCHUNK
    NSEM = NCHUNK + 1

    def body(mask_ref, part_ref, out_ref, sbuf_ref, rbufa_ref, rbufb_ref,
             csum_ref, creca_ref, crecb_ref,
             send_a, recv_a, send_b, recv_b):
        x = lax.axis_index("x")
        y = lax.axis_index("y")
        z = lax.axis_index("z")
        nbr = (1 - x, y, z)

        sbuf_ref[:, :] = part_ref[:, :].astype(jnp.bfloat16)
        ci = lax.broadcasted_iota(jnp.int32, (NCHUNK, T), 0)
        ti = lax.broadcasted_iota(jnp.int32, (NCHUNK, T), 1)
        G = (ti // R == ci).astype(jnp.bfloat16)
        gt_t = lax.broadcasted_iota(jnp.int32, (T, NCHUNK), 0)
        gt_c = lax.broadcasted_iota(jnp.int32, (T, NCHUNK), 1)
        GT = (gt_t // R == gt_c).astype(jnp.float32)
        dims = (((1,), (0,)), ((), ()))

        def chunk_sums(buf_bf16):
            return lax.dot_general(
                G, buf_bf16, dims, preferred_element_type=jnp.float32
            )

        csum_ref[:, :] = chunk_sums(sbuf_ref[:, :])

        barrier_sem = pltpu.get_barrier_semaphore()
        pl.semaphore_signal(
            barrier_sem, inc=1, device_id=nbr,
            device_id_type=pl.DeviceIdType.MESH,
        )
        pl.semaphore_wait(barrier_sem, 1)

        def round_rdmas(rbuf_ref, crec_ref, send_sems, recv_sems):
            rs = [
                pltpu.make_async_remote_copy(
                    src_ref=sbuf_ref.at[pl.ds(c * R, R)],
                    dst_ref=rbuf_ref.at[pl.ds(c * R, R)],
                    send_sem=send_sems.at[c],
                    recv_sem=recv_sems.at[c],
                    device_id=nbr,
                    device_id_type=pl.DeviceIdType.MESH,
                )
                for c in range(NCHUNK)
            ]
            rs.append(pltpu.make_async_remote_copy(
                src_ref=csum_ref,
                dst_ref=crec_ref,
                send_sem=send_sems.at[NCHUNK],
                recv_sem=recv_sems.at[NCHUNK],
                device_id=nbr,
                device_id_type=pl.DeviceIdType.MESH,
            ))
            return rs

        rdmas_a = round_rdmas(rbufa_ref, creca_ref, send_a, recv_a)
        rdmas_b = round_rdmas(rbufb_ref, crecb_ref, send_b, recv_b)
        for r in rdmas_a:
            r.start()
        for r in rdmas_b:
            r.start()
        for r in rdmas_a:
            r.wait()

        def bad_rows(recomputed, truth):
            neq = (recomputed != truth).astype(jnp.float32)
            bad_chunk = jnp.sum(neq, axis=1, keepdims=True)
            return lax.dot_general(
                GT, bad_chunk, dims, preferred_element_type=jnp.float32
            ) > 0

        bad_a = bad_rows(chunk_sums(rbufa_ref[:, :]), creca_ref[:, :])
        bad_p = bad_rows(
            chunk_sums(part_ref[:, :].astype(jnp.bfloat16)), csum_ref[:, :]
        )
        own = jnp.where(
            bad_p, sbuf_ref[:, :].astype(jnp.float32), part_ref[:, :]
        )
        m = mask_ref[:, :] > 0
        pre = jnp.where(m, own, rbufa_ref[:, :].astype(jnp.float32))

        for r in rdmas_b:
            r.wait()
        out_ref[:, :] = jnp.where(
            jnp.logical_and(bad_a, jnp.logical_not(m)),
            rbufb_ref[:, :].astype(jnp.float32),
            pre,
        )

        @functools.partial(
            pl.run_scoped, second_barrier=pltpu.SemaphoreType.REGULAR
        )
        def _(second_barrier):
            pl.semaphore_signal(
                second_barrier, inc=1, device_id=nbr,
                device_id_type=pl.DeviceIdType.MESH,
            )
            pl.semaphore_wait(second_barrier, 1)

    return pl.pallas_call(
        body,
        out_shape=jax.ShapeDtypeStruct((T, D), jnp.float32),
        in_specs=[
            pl.BlockSpec(memory_space=pltpu.VMEM),
            pl.BlockSpec(memory_space=pltpu.VMEM),
        ],
        out_specs=pl.BlockSpec(memory_space=pltpu.VMEM),
        scratch_shapes=[
            pltpu.VMEM((T, D), jnp.bfloat16),
            pltpu.VMEM((T, D), jnp.bfloat16),
            pltpu.VMEM((T, D), jnp.bfloat16),
            pltpu.VMEM((NCHUNK, D), jnp.float32),
            pltpu.VMEM((NCHUNK, D), jnp.float32),
            pltpu.VMEM((NCHUNK, D), jnp.float32),
            pltpu.SemaphoreType.DMA((NSEM,)),
            pltpu.SemaphoreType.DMA((NSEM,)),
            pltpu.SemaphoreType.DMA((NSEM,)),
            pltpu.SemaphoreType.DMA((NSEM,)),
        ],
        compiler_params=pltpu.CompilerParams(collective_id=0),
    )(mask, part)
